# baseline (speedup 1.0000x reference)
import functools

import jax
import jax.numpy as jnp
from jax import lax
from jax.experimental import pallas as pl
from jax.experimental.pallas import tpu as pltpu
from jax.experimental.pallas import tpu_sc as plsc

MU_CONST = 3.5
HIDDEN_DIM = 32
CHUNK = 128


def _make_lfm(batch: int):
    info = plsc.get_sparse_core_info()
    nc, ns = info.num_cores, info.num_subcores
    nw = nc * ns
    assert batch % (nw * CHUNK) == 0
    bpw = batch // nw
    nchunk = bpw // CHUNK

    mesh = plsc.VectorSubcoreMesh(core_axis_name="c", subcore_axis_name="s")

    @functools.partial(
        pl.kernel,
        out_type=jax.ShapeDtypeStruct((batch,), jnp.float32),
        mesh=mesh,
        compiler_params=pltpu.CompilerParams(
            needs_layout_passes=False, use_tc_tiling_on_sc=False),
        scratch_types=[
            pltpu.VMEM((nchunk, CHUNK), jnp.int32),
            pltpu.VMEM((nchunk, CHUNK), jnp.int32),
            pltpu.VMEM((bpw, HIDDEN_DIM), jnp.float32),
            pltpu.VMEM((bpw, HIDDEN_DIM), jnp.float32),
            pltpu.VMEM((bpw,), jnp.float32),
            pltpu.VMEM((bpw,), jnp.float32),
            pltpu.VMEM((bpw,), jnp.float32),
            pltpu.SemaphoreType.DMA,
        ],
    )
    def lfm(uidx_hbm, iidx_hbm, ue_hbm, ie_hbm, bu_hbm, bi_hbm, out_hbm,
            uidx_v, iidx_v, p_v, q_v, bu_v, bi_v, out_v, sem):
        wid = lax.axis_index("s") * nc + lax.axis_index("c")
        base = wid * bpw

        pltpu.sync_copy(uidx_hbm.at[pl.ds(wid * nchunk, nchunk)], uidx_v)
        pltpu.sync_copy(iidx_hbm.at[pl.ds(wid * nchunk, nchunk)], iidx_v)

        copies = []
        for j in range(nchunk):
            dst = pl.ds(j * CHUNK, CHUNK)
            copies.append(pltpu.async_copy(
                ue_hbm.at[uidx_v.at[j]], p_v.at[dst], sem))
            copies.append(pltpu.async_copy(
                ie_hbm.at[iidx_v.at[j]], q_v.at[dst], sem))
            copies.append(pltpu.async_copy(
                bu_hbm.at[uidx_v.at[j]], bu_v.at[dst], sem))
            copies.append(pltpu.async_copy(
                bi_hbm.at[iidx_v.at[j]], bi_v.at[dst], sem))
        for c in copies:
            c.wait()

        lane = lax.iota(jnp.int32, 16)

        def body(g, carry):
            rows = g * 16 + lane
            sl = pl.ds(g * 16, 16)
            acc = bu_v[sl] + bi_v[sl] + MU_CONST
            for h in range(HIDDEN_DIM):
                col = jnp.full((16,), h, jnp.int32)
                acc += (plsc.load_gather(p_v, [rows, col])
                        * plsc.load_gather(q_v, [rows, col]))
            out_v[sl] = acc
            return carry

        lax.fori_loop(0, bpw // 16, body, 0)

        pltpu.sync_copy(out_v, out_hbm.at[pl.ds(base, bpw)])

    return lfm


def kernel(user_indexs, item_indexs, user_embed, item_embed, bias_u, bias_i):
    batch = user_indexs.shape[0]
    info = plsc.get_sparse_core_info()
    nw = info.num_cores * info.num_subcores
    u2 = user_indexs.astype(jnp.int32).reshape(batch // CHUNK, CHUNK)
    i2 = item_indexs.astype(jnp.int32).reshape(batch // CHUNK, CHUNK)
    out = _make_lfm(batch)(u2, i2, user_embed, item_embed,
                           bias_u.reshape(-1), bias_i.reshape(-1))
    return out.reshape(batch, 1)
